# baseline (speedup 1.0000x reference)
import functools

import jax
import jax.numpy as jnp
from jax.experimental import pallas as pl
from jax.experimental.pallas import tpu as pltpu


def _round_up(x, m):
    return ((x + m - 1) // m) * m


def _pad2d(x, rows, cols):
    r, c = x.shape
    if r == rows and c == cols:
        return x
    return jnp.pad(x, ((0, rows - r), (0, cols - c)))


def _fused_kernel(miu_ref, sigma_ref, wm_ref, ws_ref, adj1_ref, adj2_ref,
                  m_ref, s_ref, im_ref, is_ref, *, gamma):
    j = pl.program_id(1)

    @pl.when(j == 0)
    def _():
        miu = miu_ref[...]
        sigma = sigma_ref[...]
        att = jnp.exp(-gamma * sigma)
        m_in = (miu * att).astype(jnp.bfloat16)
        s_in = (sigma * att * att).astype(jnp.bfloat16)
        im_ref[...] = jnp.dot(m_in, wm_ref[...],
                              preferred_element_type=jnp.float32
                              ).astype(im_ref.dtype)
        is_ref[...] = jnp.dot(s_in, ws_ref[...],
                              preferred_element_type=jnp.float32
                              ).astype(is_ref.dtype)

    a1 = adj1_ref[...].astype(jnp.bfloat16)
    a2 = adj2_ref[...].astype(jnp.bfloat16)
    m_ref[...] = jnp.dot(a1, im_ref[...], preferred_element_type=jnp.float32)
    s_ref[...] = jnp.dot(a2, is_ref[...], preferred_element_type=jnp.float32)


def kernel(previous_miu, previous_sigma, weight_miu, weight_sigma,
           adj_norm1, adj_norm2):
    gamma = 1.0
    n, f_in = previous_miu.shape
    f_out = weight_miu.shape[1]
    out_dtype = previous_miu.dtype

    f_pad = _round_up(f_out, 128)
    tm = 512 if n >= 1024 else _round_up(n, 128)
    n_pad = _round_up(n, 2 * tm)
    panels_per_core = n_pad // tm // 2

    miu_p = _pad2d(previous_miu, n_pad, f_in)
    sigma_p = _pad2d(previous_sigma, n_pad, f_in)
    wm_p = _pad2d(weight_miu, f_in, f_pad).astype(jnp.bfloat16)
    ws_p = _pad2d(weight_sigma, f_in, f_pad).astype(jnp.bfloat16)
    adj1_p = _pad2d(adj_norm1, n_pad, n_pad)
    adj2_p = _pad2d(adj_norm2, n_pad, n_pad)

    adj_bytes = jnp.dtype(adj_norm1.dtype).itemsize
    m_out, s_out = pl.pallas_call(
        functools.partial(_fused_kernel, gamma=float(gamma)),
        grid=(2, panels_per_core),
        out_shape=(jax.ShapeDtypeStruct((n_pad, f_pad), out_dtype),
                   jax.ShapeDtypeStruct((n_pad, f_pad), out_dtype)),
        in_specs=[pl.BlockSpec((n_pad, f_in), lambda i, j: (0, 0)),
                  pl.BlockSpec((n_pad, f_in), lambda i, j: (0, 0)),
                  pl.BlockSpec((f_in, f_pad), lambda i, j: (0, 0)),
                  pl.BlockSpec((f_in, f_pad), lambda i, j: (0, 0)),
                  pl.BlockSpec((tm, n_pad),
                               lambda i, j, p=panels_per_core: (i * p + j, 0)),
                  pl.BlockSpec((tm, n_pad),
                               lambda i, j, p=panels_per_core: (i * p + j, 0))],
        out_specs=(pl.BlockSpec((tm, f_pad),
                                lambda i, j, p=panels_per_core: (i * p + j, 0)),
                   pl.BlockSpec((tm, f_pad),
                                lambda i, j, p=panels_per_core: (i * p + j, 0))),
        scratch_shapes=[pltpu.VMEM((n_pad, f_pad), jnp.bfloat16),
                        pltpu.VMEM((n_pad, f_pad), jnp.bfloat16)],
        compiler_params=pltpu.CompilerParams(
            dimension_semantics=("parallel", "arbitrary")),
        cost_estimate=pl.CostEstimate(
            flops=int(4 * n_pad * n_pad * f_pad + 4 * n_pad * f_in * f_pad),
            transcendentals=int(n_pad * f_in),
            bytes_accessed=int(2 * n_pad * n_pad * adj_bytes
                               + 2 * n_pad * f_in * 4
                               + 2 * n_pad * f_pad * 4)),
    )(miu_p, sigma_p, wm_p, ws_p, adj1_p, adj2_p)

    if n_pad == n and f_pad == f_out:
        return m_out, s_out
    return m_out[:n, :f_out], s_out[:n, :f_out]


# speedup vs baseline: 1.0011x; 1.0011x over previous
import functools

import jax
import jax.numpy as jnp
from jax.experimental import pallas as pl
from jax.experimental.pallas import tpu as pltpu


def _round_up(x, m):
    return ((x + m - 1) // m) * m


def _pad2d(x, rows, cols):
    r, c = x.shape
    if r == rows and c == cols:
        return x
    return jnp.pad(x, ((0, rows - r), (0, cols - c)))


def _fused_kernel(miu_ref, sigma_ref, wm_ref, ws_ref, adj1_ref, adj2_ref,
                  m_ref, s_ref, im_ref, is_ref, *, gamma):
    j = pl.program_id(1)

    @pl.when(j == 0)
    def _():
        miu = miu_ref[...]
        sigma = sigma_ref[...]
        att = jnp.exp(-gamma * sigma)
        m_in = (miu * att).astype(jnp.bfloat16)
        s_in = (sigma * att * att).astype(jnp.bfloat16)
        im_ref[...] = jnp.dot(m_in, wm_ref[...],
                              preferred_element_type=jnp.float32
                              ).astype(im_ref.dtype)
        is_ref[...] = jnp.dot(s_in, ws_ref[...],
                              preferred_element_type=jnp.float32
                              ).astype(is_ref.dtype)

    a1 = adj1_ref[...].astype(jnp.bfloat16)
    a2 = adj2_ref[...].astype(jnp.bfloat16)
    m_ref[...] = jnp.dot(a1, im_ref[...], preferred_element_type=jnp.float32)
    s_ref[...] = jnp.dot(a2, is_ref[...], preferred_element_type=jnp.float32)


def kernel(previous_miu, previous_sigma, weight_miu, weight_sigma,
           adj_norm1, adj_norm2):
    gamma = 1.0
    n, f_in = previous_miu.shape
    f_out = weight_miu.shape[1]
    out_dtype = previous_miu.dtype

    f_pad = _round_up(f_out, 128)
    tm = 256 if n >= 512 else _round_up(n, 128)
    n_pad = _round_up(n, 2 * tm)
    panels_per_core = n_pad // tm // 2

    miu_p = _pad2d(previous_miu, n_pad, f_in)
    sigma_p = _pad2d(previous_sigma, n_pad, f_in)
    wm_p = _pad2d(weight_miu, f_in, f_pad).astype(jnp.bfloat16)
    ws_p = _pad2d(weight_sigma, f_in, f_pad).astype(jnp.bfloat16)
    adj1_p = _pad2d(adj_norm1, n_pad, n_pad)
    adj2_p = _pad2d(adj_norm2, n_pad, n_pad)

    adj_bytes = jnp.dtype(adj_norm1.dtype).itemsize
    m_out, s_out = pl.pallas_call(
        functools.partial(_fused_kernel, gamma=float(gamma)),
        grid=(2, panels_per_core),
        out_shape=(jax.ShapeDtypeStruct((n_pad, f_pad), out_dtype),
                   jax.ShapeDtypeStruct((n_pad, f_pad), out_dtype)),
        in_specs=[pl.BlockSpec((n_pad, f_in), lambda i, j: (0, 0)),
                  pl.BlockSpec((n_pad, f_in), lambda i, j: (0, 0)),
                  pl.BlockSpec((f_in, f_pad), lambda i, j: (0, 0)),
                  pl.BlockSpec((f_in, f_pad), lambda i, j: (0, 0)),
                  pl.BlockSpec((tm, n_pad),
                               lambda i, j, p=panels_per_core: (i * p + j, 0)),
                  pl.BlockSpec((tm, n_pad),
                               lambda i, j, p=panels_per_core: (i * p + j, 0))],
        out_specs=(pl.BlockSpec((tm, f_pad),
                                lambda i, j, p=panels_per_core: (i * p + j, 0)),
                   pl.BlockSpec((tm, f_pad),
                                lambda i, j, p=panels_per_core: (i * p + j, 0))),
        scratch_shapes=[pltpu.VMEM((n_pad, f_pad), jnp.bfloat16),
                        pltpu.VMEM((n_pad, f_pad), jnp.bfloat16)],
        compiler_params=pltpu.CompilerParams(
            dimension_semantics=("parallel", "arbitrary")),
        cost_estimate=pl.CostEstimate(
            flops=int(4 * n_pad * n_pad * f_pad + 4 * n_pad * f_in * f_pad),
            transcendentals=int(n_pad * f_in),
            bytes_accessed=int(2 * n_pad * n_pad * adj_bytes
                               + 2 * n_pad * f_in * 4
                               + 2 * n_pad * f_pad * 4)),
    )(miu_p, sigma_p, wm_p, ws_p, adj1_p, adj2_p)

    if n_pad == n and f_pad == f_out:
        return m_out, s_out
    return m_out[:n, :f_out], s_out[:n, :f_out]


# speedup vs baseline: 1.0637x; 1.0625x over previous
import functools

import jax
import jax.numpy as jnp
from jax.experimental import pallas as pl
from jax.experimental.pallas import tpu as pltpu


def _round_up(x, m):
    return ((x + m - 1) // m) * m


def _pad2d(x, rows, cols):
    r, c = x.shape
    if r == rows and c == cols:
        return x
    return jnp.pad(x, ((0, rows - r), (0, cols - c)))


def _fused_kernel(miu_ref, sigma_ref, wm_ref, ws_ref, adj1_ref, adj2_ref,
                  m_ref, s_ref, im_ref, is_ref, *, gamma):
    j = pl.program_id(1)

    @pl.when(j == 0)
    def _():
        miu = miu_ref[...]
        sigma = sigma_ref[...]
        att = jnp.exp(-gamma * sigma)
        m_in = (miu * att).astype(jnp.bfloat16)
        s_in = (sigma * att * att).astype(jnp.bfloat16)
        wm = wm_ref[...].astype(jnp.bfloat16)
        ws = ws_ref[...].astype(jnp.bfloat16)
        im_ref[...] = jnp.dot(m_in, wm,
                              preferred_element_type=jnp.float32
                              ).astype(im_ref.dtype)
        is_ref[...] = jnp.dot(s_in, ws,
                              preferred_element_type=jnp.float32
                              ).astype(is_ref.dtype)

    a1 = adj1_ref[...].astype(jnp.bfloat16)
    a2 = adj2_ref[...].astype(jnp.bfloat16)
    m_ref[...] = jnp.dot(a1, im_ref[...], preferred_element_type=jnp.float32)
    s_ref[...] = jnp.dot(a2, is_ref[...], preferred_element_type=jnp.float32)


def kernel(previous_miu, previous_sigma, weight_miu, weight_sigma,
           adj_norm1, adj_norm2):
    gamma = 1.0
    n, f_in = previous_miu.shape
    f_out = weight_miu.shape[1]
    out_dtype = previous_miu.dtype

    f_pad = _round_up(f_out, 128)
    tm = 256 if n >= 512 else _round_up(n, 128)
    n_pad = _round_up(n, 2 * tm)
    panels_per_core = n_pad // tm // 2

    miu_p = _pad2d(previous_miu, n_pad, f_in)
    sigma_p = _pad2d(previous_sigma, n_pad, f_in)
    wm_p = _pad2d(weight_miu, f_in, f_pad)
    ws_p = _pad2d(weight_sigma, f_in, f_pad)
    adj1_p = _pad2d(adj_norm1, n_pad, n_pad)
    adj2_p = _pad2d(adj_norm2, n_pad, n_pad)

    adj_bytes = jnp.dtype(adj_norm1.dtype).itemsize
    m_out, s_out = pl.pallas_call(
        functools.partial(_fused_kernel, gamma=float(gamma)),
        grid=(2, panels_per_core),
        out_shape=(jax.ShapeDtypeStruct((n_pad, f_pad), out_dtype),
                   jax.ShapeDtypeStruct((n_pad, f_pad), out_dtype)),
        in_specs=[pl.BlockSpec((n_pad, f_in), lambda i, j: (0, 0)),
                  pl.BlockSpec((n_pad, f_in), lambda i, j: (0, 0)),
                  pl.BlockSpec((f_in, f_pad), lambda i, j: (0, 0)),
                  pl.BlockSpec((f_in, f_pad), lambda i, j: (0, 0)),
                  pl.BlockSpec((tm, n_pad),
                               lambda i, j, p=panels_per_core: (i * p + j, 0)),
                  pl.BlockSpec((tm, n_pad),
                               lambda i, j, p=panels_per_core: (i * p + j, 0))],
        out_specs=(pl.BlockSpec((tm, f_pad),
                                lambda i, j, p=panels_per_core: (i * p + j, 0)),
                   pl.BlockSpec((tm, f_pad),
                                lambda i, j, p=panels_per_core: (i * p + j, 0))),
        scratch_shapes=[pltpu.VMEM((n_pad, f_pad), jnp.bfloat16),
                        pltpu.VMEM((n_pad, f_pad), jnp.bfloat16)],
        compiler_params=pltpu.CompilerParams(
            dimension_semantics=("parallel", "arbitrary")),
        cost_estimate=pl.CostEstimate(
            flops=int(4 * n_pad * n_pad * f_pad + 4 * n_pad * f_in * f_pad),
            transcendentals=int(n_pad * f_in),
            bytes_accessed=int(2 * n_pad * n_pad * adj_bytes
                               + 2 * n_pad * f_in * 4
                               + 2 * n_pad * f_pad * 4)),
    )(miu_p, sigma_p, wm_p, ws_p, adj1_p, adj2_p)

    if n_pad == n and f_pad == f_out:
        return m_out, s_out
    return m_out[:n, :f_out], s_out[:n, :f_out]
